# baseline (speedup 1.0000x reference)
import functools

import jax
import jax.numpy as jnp
from jax import lax
from jax.experimental import pallas as pl
from jax.experimental.pallas import tpu as pltpu
from jax.experimental.pallas import tpu_sc as plsc

_B, _C, _H, _W = 8, 19, 512, 512
_NC, _NS = 2, 16
_NW = _NC * _NS
_BSC = 5
_RPW = (_BSC * _H) // _NW
_SR = 8
_NSLAB = _RPW // _SR
_QW = 128
_NQ = _W // _QW
_NBUF = _NQ

_HB = 64
_mesh = plsc.VectorSubcoreMesh(core_axis_name="c", subcore_axis_name="s")


@functools.partial(
    pl.kernel,
    out_type=jax.ShapeDtypeStruct((_NW, 16), jnp.float32),
    mesh=_mesh,
    compiler_params=pltpu.CompilerParams(needs_layout_passes=False),
    scratch_types=[
        pltpu.VMEM((_NBUF, _C, _SR, _QW), jnp.float32),
        pltpu.VMEM((2, _SR, _W), jnp.int32),
        pltpu.VMEM((2, _SR, _W), jnp.int32),
        pltpu.VMEM((16,), jnp.float32),
        pltpu.SemaphoreType.DMA((_NBUF,)),
        pltpu.SemaphoreType.DMA((2,)),
        pltpu.SemaphoreType.DMA((2,)),
    ],
)
def _partials(z_hbm, l_hbm, c_hbm, out_hbm,
              zq, lb, cb, acc_v, sem_z, sem_l, sem_c):
    cid = lax.axis_index("c")
    sid = lax.axis_index("s")
    wid = sid * _NC + cid
    row_base = wid * _RPW
    iota = lax.iota(jnp.int32, 16)

    def rowspec(s):
        g = row_base + s * _SR
        return g // _H, g % _H

    def z_desc(s, k):
        b, h0 = rowspec(s)
        return pltpu.make_async_copy(
            z_hbm.at[b, :, pl.ds(h0, _SR), pl.ds(k * _QW, _QW)],
            zq.at[k], sem_z.at[k])

    def l_desc(s):
        b, h0 = rowspec(s)
        return pltpu.make_async_copy(
            l_hbm.at[b, pl.ds(h0, _SR), :], lb.at[s % 2], sem_l.at[s % 2])

    def c_desc(s):
        b, h0 = rowspec(s)
        return pltpu.make_async_copy(
            c_hbm.at[b, pl.ds(h0, _SR), :], cb.at[s % 2], sem_c.at[s % 2])

    l_desc(0).start()
    c_desc(0).start()
    for k in range(_NBUF):
        z_desc(0, k).start()

    def slab_body(s, total):
        @pl.when(s + 1 < _NSLAB)
        def _():
            l_desc(s + 1).start()
            c_desc(s + 1).start()
        l_desc(s).wait()
        c_desc(s).wait()

        for k in range(_NQ):
            z_desc(s, k).wait()
            zref = zq.at[k]

            def body(i, acc):
                hl = i >> 3
                jj = i & 7
                lv = lb[s % 2, hl, pl.ds(k * _QW + jj * 16, 16)]
                cv = cb[s % 2, hl, pl.ds(k * _QW + jj * 16, 16)]
                hvec = jnp.full((16,), hl, jnp.int32)
                wvec = jj * 16 + iota
                gv = plsc.load_gather(zref, [lv, hvec, wvec])
                return acc + jnp.where(cv > 0, gv,
                                       jnp.zeros((16,), jnp.float32))

            qacc = lax.fori_loop(0, (_SR * _QW) // 16, body,
                                 jnp.zeros((16,), jnp.float32))
            total = total + qacc

            @pl.when(s + 1 < _NSLAB)
            def _():
                z_desc(s + 1, k).start()
        return total

    total = lax.fori_loop(0, _NSLAB, slab_body,
                          jnp.zeros((16,), jnp.float32))
    acc_v[...] = total
    pltpu.sync_copy(acc_v, out_hbm.at[wid])


def _tc_body(z_ref, l_ref, c_ref, out_ref):
    bi = pl.program_id(0)
    hi = pl.program_id(1)

    @pl.when((bi == 0) & (hi == 0))
    def _():
        out_ref[...] = jnp.zeros((8, 128), jnp.float32)

    lv = l_ref[0]
    cv = c_ref[0]
    acc = jnp.zeros((_HB, _W), jnp.float32)
    for c in range(_C):
        acc = acc + jnp.where(lv == c, z_ref[0, c], 0.0)
    acc = jnp.where(cv > 0, acc, 0.0)
    out_ref[...] += jnp.sum(acc.reshape(8, 8, 4, 128), axis=(1, 2))


def _tc_partial(z_tc, l_tc, c_tc):
    nb = _B - _BSC
    return pl.pallas_call(
        _tc_body,
        grid=(nb, _H // _HB),
        in_specs=[
            pl.BlockSpec((1, _C, _HB, _W),
                         lambda bi, hi: (bi + _BSC, 0, hi, 0)),
            pl.BlockSpec((1, _HB, _W), lambda bi, hi: (bi + _BSC, hi, 0)),
            pl.BlockSpec((1, _HB, _W), lambda bi, hi: (bi + _BSC, hi, 0)),
        ],
        out_specs=pl.BlockSpec((8, 128), lambda bi, hi: (0, 0)),
        out_shape=jax.ShapeDtypeStruct((8, 128), jnp.float32),
        compiler_params=pltpu.CompilerParams(
            dimension_semantics=("arbitrary", "arbitrary")),
    )(z_tc, l_tc, c_tc)


def _finish_body(p_ref, t_ref, out_ref):
    s = jnp.sum(p_ref[...]) + jnp.sum(t_ref[...])
    out_ref[...] = jnp.full((8, 128), s, jnp.float32)


def _finish(parts, tc_part):
    return pl.pallas_call(
        _finish_body,
        out_shape=jax.ShapeDtypeStruct((8, 128), jnp.float32),
    )(parts, tc_part)


def kernel(z, condition, l):
    cf = condition.astype(jnp.int32)
    parts = _partials(z, l, cf)
    tc_part = _tc_partial(z, l, cf)
    out = _finish(parts, tc_part)
    return out[0, 0]


# speedup vs baseline: 1.0293x; 1.0293x over previous
import functools

import jax
import jax.numpy as jnp
from jax import lax
from jax.experimental import pallas as pl
from jax.experimental.pallas import tpu as pltpu
from jax.experimental.pallas import tpu_sc as plsc

_B, _C, _H, _W = 8, 19, 512, 512
_NC, _NS = 2, 16
_NW = _NC * _NS
_BSC = 4
_RPW = (_BSC * _H) // _NW
_SR = 8
_NSLAB = _RPW // _SR
_QW = 128
_NQ = _W // _QW
_NBUF = _NQ

_HB = 64
_mesh = plsc.VectorSubcoreMesh(core_axis_name="c", subcore_axis_name="s")


@functools.partial(
    pl.kernel,
    out_type=jax.ShapeDtypeStruct((_NW, 16), jnp.float32),
    mesh=_mesh,
    compiler_params=pltpu.CompilerParams(needs_layout_passes=False),
    scratch_types=[
        pltpu.VMEM((_NBUF, _C, _SR, _QW), jnp.float32),
        pltpu.VMEM((2, _SR, _W), jnp.int32),
        pltpu.VMEM((2, _SR, _W), jnp.int32),
        pltpu.VMEM((16,), jnp.float32),
        pltpu.SemaphoreType.DMA((_NBUF,)),
        pltpu.SemaphoreType.DMA((2,)),
        pltpu.SemaphoreType.DMA((2,)),
    ],
)
def _partials(z_hbm, l_hbm, c_hbm, out_hbm,
              zq, lb, cb, acc_v, sem_z, sem_l, sem_c):
    cid = lax.axis_index("c")
    sid = lax.axis_index("s")
    wid = sid * _NC + cid
    row_base = wid * _RPW
    iota = lax.iota(jnp.int32, 16)

    def rowspec(s):
        g = row_base + s * _SR
        return g // _H, g % _H

    def z_desc(s, k):
        b, h0 = rowspec(s)
        return pltpu.make_async_copy(
            z_hbm.at[b, :, pl.ds(h0, _SR), pl.ds(k * _QW, _QW)],
            zq.at[k], sem_z.at[k])

    def l_desc(s):
        b, h0 = rowspec(s)
        return pltpu.make_async_copy(
            l_hbm.at[b, pl.ds(h0, _SR), :], lb.at[s % 2], sem_l.at[s % 2])

    def c_desc(s):
        b, h0 = rowspec(s)
        return pltpu.make_async_copy(
            c_hbm.at[b, pl.ds(h0, _SR), :], cb.at[s % 2], sem_c.at[s % 2])

    l_desc(0).start()
    c_desc(0).start()
    for k in range(_NBUF):
        z_desc(0, k).start()

    def slab_body(s, total):
        @pl.when(s + 1 < _NSLAB)
        def _():
            l_desc(s + 1).start()
            c_desc(s + 1).start()
        l_desc(s).wait()
        c_desc(s).wait()

        for k in range(_NQ):
            z_desc(s, k).wait()
            zref = zq.at[k]

            def body(i, acc):
                hl = i >> 3
                jj = i & 7
                lv = lb[s % 2, hl, pl.ds(k * _QW + jj * 16, 16)]
                cv = cb[s % 2, hl, pl.ds(k * _QW + jj * 16, 16)]
                hvec = jnp.full((16,), hl, jnp.int32)
                wvec = jj * 16 + iota
                gv = plsc.load_gather(zref, [lv, hvec, wvec])
                return acc + jnp.where(cv > 0, gv,
                                       jnp.zeros((16,), jnp.float32))

            qacc = lax.fori_loop(0, (_SR * _QW) // 16, body,
                                 jnp.zeros((16,), jnp.float32))
            total = total + qacc

            @pl.when(s + 1 < _NSLAB)
            def _():
                z_desc(s + 1, k).start()
        return total

    total = lax.fori_loop(0, _NSLAB, slab_body,
                          jnp.zeros((16,), jnp.float32))
    acc_v[...] = total
    pltpu.sync_copy(acc_v, out_hbm.at[wid])


def _tc_body(z_ref, l_ref, c_ref, out_ref):
    bi = pl.program_id(0)
    hi = pl.program_id(1)

    @pl.when((bi == 0) & (hi == 0))
    def _():
        out_ref[...] = jnp.zeros((8, 128), jnp.float32)

    lv = l_ref[0]
    cv = c_ref[0]
    acc = jnp.zeros((_HB, _W), jnp.float32)
    for c in range(_C):
        acc = acc + jnp.where(lv == c, z_ref[0, c], 0.0)
    acc = jnp.where(cv > 0, acc, 0.0)
    out_ref[...] += jnp.sum(acc.reshape(8, 8, 4, 128), axis=(1, 2))


def _tc_partial(z_tc, l_tc, c_tc):
    nb = _B - _BSC
    return pl.pallas_call(
        _tc_body,
        grid=(nb, _H // _HB),
        in_specs=[
            pl.BlockSpec((1, _C, _HB, _W),
                         lambda bi, hi: (bi + _BSC, 0, hi, 0)),
            pl.BlockSpec((1, _HB, _W), lambda bi, hi: (bi + _BSC, hi, 0)),
            pl.BlockSpec((1, _HB, _W), lambda bi, hi: (bi + _BSC, hi, 0)),
        ],
        out_specs=pl.BlockSpec((8, 128), lambda bi, hi: (0, 0)),
        out_shape=jax.ShapeDtypeStruct((8, 128), jnp.float32),
        compiler_params=pltpu.CompilerParams(
            dimension_semantics=("arbitrary", "arbitrary")),
    )(z_tc, l_tc, c_tc)


def _finish_body(p_ref, t_ref, out_ref):
    s = jnp.sum(p_ref[...]) + jnp.sum(t_ref[...])
    out_ref[...] = jnp.full((8, 128), s, jnp.float32)


def _finish(parts, tc_part):
    return pl.pallas_call(
        _finish_body,
        out_shape=jax.ShapeDtypeStruct((8, 128), jnp.float32),
    )(parts, tc_part)


def kernel(z, condition, l):
    cf = condition.astype(jnp.int32)
    parts = _partials(z, l, cf)
    tc_part = _tc_partial(z, l, cf)
    out = _finish(parts, tc_part)
    return out[0, 0]
